# Initial kernel scaffold; baseline (speedup 1.0000x reference)
import jax
import jax.numpy as jnp
from jax.experimental import pallas as pl


def kernel(wq, wk, wv, wo, cen):
    raise NotImplementedError("write your pallas kernel here")


# speedup vs baseline: 4.3020x; 4.3020x over previous
import math
import functools

import jax
import jax.numpy as jnp
from jax import lax
from jax.experimental import pallas as pl
from jax.experimental.pallas import tpu as pltpu


def _attn_kernel(cen_ref, wq_ref, wkv_ref, wo_ref, y_ref, st_ref, *,
                 num_heads, hidden, hid8, H, W, inv_sqrt_area):
    HW = H * W
    hslice = hid8 // num_heads

    cen = cen_ref[...]
    cen_bf = cen.astype(jnp.bfloat16)

    pix = lax.broadcasted_iota(jnp.int32, cen.shape, 1)
    row = lax.shift_right_logical(pix, 5)
    col = lax.bitwise_and(pix, W - 1)

    q_all = jnp.dot(wq_ref[...], cen_bf, preferred_element_type=jnp.float32)

    def _roll(x, s):
        return jnp.roll(x, s, axis=1)

    kvs = []
    for h in range(num_heads):
        d = (1, 2)[h]
        if d == 1:
            rneg = lambda x: jnp.where(row == 0, _roll(x, -W), _roll(x, W))
            rpos = lambda x: jnp.where(row == H - 1, _roll(x, W), _roll(x, -W))
            cn = jnp.where(col == 0, _roll(cen, -1), _roll(cen, 1))
            cp = jnp.where(col == W - 1, _roll(cen, 1), _roll(cen, -1))
        else:
            rneg = lambda x: jnp.where(
                row == 0, _roll(x, -2 * W),
                jnp.where(row == 1, x, _roll(x, 2 * W)))
            rpos = lambda x: jnp.where(
                row == H - 2, x,
                jnp.where(row == H - 1, _roll(x, 2 * W), _roll(x, -2 * W)))
            cn = jnp.where(col == 0, _roll(cen, -2),
                           jnp.where(col == 1, cen, _roll(cen, 2)))
            cp = jnp.where(col == W - 2, cen,
                           jnp.where(col == W - 1, _roll(cen, 2),
                                     _roll(cen, -2)))
        imgs = (rneg(cn), rneg(cen), rneg(cp), cn, cp,
                rpos(cn), rpos(cen), rpos(cp))
        sur = jnp.concatenate([im - cen for im in imgs],
                              axis=0).astype(jnp.bfloat16)
        kvs.append(jnp.dot(wkv_ref[h], sur,
                           preferred_element_type=jnp.float32))

    outs = []
    for n in range(num_heads):
        lo = n * hslice
        k = jnp.concatenate([kv[lo:lo + hslice] for kv in kvs], axis=0)
        v = jnp.concatenate([kv[hid8 + lo:hid8 + lo + hslice] for kv in kvs],
                            axis=0)
        q = q_all[n * hidden:(n + 1) * hidden]

        qn = q * lax.rsqrt(jnp.maximum(
            jnp.sum(q * q, axis=-1, keepdims=True), 1e-24))
        kn = k * lax.rsqrt(jnp.maximum(
            jnp.sum(k * k, axis=-1, keepdims=True), 1e-24))

        s = lax.dot_general(qn, kn, (((1,), (1,)), ((), ())),
                            preferred_element_type=jnp.float32) * inv_sqrt_area

        rsum = jnp.sum(s, axis=-1, keepdims=True)
        mu = jnp.sum(rsum, axis=0, keepdims=True) / (hidden * hid8)
        c = s - mu
        rsq = jnp.sum(c * c, axis=-1, keepdims=True)
        var = jnp.sum(rsq, axis=0, keepdims=True) / (hidden * hid8)
        s = c * lax.rsqrt(var + 1e-5)

        s = s - jnp.max(s, axis=-1, keepdims=True)
        e = jnp.exp(s)
        p = e / jnp.sum(e, axis=-1, keepdims=True)

        outs.append(jnp.dot(p, v, preferred_element_type=jnp.float32))

    o_all = jnp.concatenate(outs, axis=0)
    y = jnp.dot(wo_ref[...], o_all, preferred_element_type=jnp.float32)

    y_ref[...] = y
    st_ref[...] = jnp.concatenate(
        [jnp.sum(y, axis=1, keepdims=True),
         jnp.sum(y * y, axis=1, keepdims=True)], axis=1)


def _bn_relu_kernel(y_ref, st_ref, o_ref, *, count):
    tot = jnp.sum(st_ref[...], axis=0)
    inv = 1.0 / count
    mu = tot[:, 0:1] * inv
    var = tot[:, 1:2] * inv - mu * mu
    scale = lax.rsqrt(var + 1e-5)
    o_ref[...] = jnp.maximum((y_ref[...] - mu) * scale, 0.0)


def kernel(wq, wk, wv, wo, cen):
    B, C, H, W = cen.shape
    NH, hidden = wq.shape[0], wq.shape[1]
    hid8 = wk.shape[1]
    tra = NH * hidden
    out_ch = wo.shape[0]
    HW = H * W

    cen_flat = cen.astype(jnp.float32).reshape(B, C, HW)

    wq_perm = wq.transpose(1, 0, 2).reshape(tra, C).astype(jnp.bfloat16)
    wkv = jnp.stack([jnp.concatenate([wk[h], wv[h]], axis=0)
                     for h in range(NH)]).astype(jnp.bfloat16)

    attn = functools.partial(
        _attn_kernel, num_heads=NH, hidden=hidden, hid8=hid8, H=H, W=W,
        inv_sqrt_area=1.0 / math.sqrt(HW))

    y_pre, stats = pl.pallas_call(
        attn,
        out_shape=(jax.ShapeDtypeStruct((B, out_ch, HW), jnp.float32),
                   jax.ShapeDtypeStruct((B, out_ch, 2), jnp.float32)),
        grid=(B,),
        in_specs=[
            pl.BlockSpec((None, C, HW), lambda b: (b, 0, 0)),
            pl.BlockSpec((tra, C), lambda b: (0, 0)),
            pl.BlockSpec((NH, 2 * hid8, 8 * C), lambda b: (0, 0, 0)),
            pl.BlockSpec((out_ch, tra), lambda b: (0, 0)),
        ],
        out_specs=(pl.BlockSpec((None, out_ch, HW), lambda b: (b, 0, 0)),
                   pl.BlockSpec((None, out_ch, 2), lambda b: (b, 0, 0))),
        compiler_params=pltpu.CompilerParams(
            dimension_semantics=("parallel",)),
    )(cen_flat, wq_perm, wkv, wo.astype(jnp.float32))

    bn = functools.partial(_bn_relu_kernel, count=float(B * HW))
    y = pl.pallas_call(
        bn,
        out_shape=jax.ShapeDtypeStruct((B, out_ch, HW), jnp.float32),
        grid=(B,),
        in_specs=[
            pl.BlockSpec((None, out_ch, HW), lambda b: (b, 0, 0)),
            pl.BlockSpec((B, out_ch, 2), lambda b: (0, 0, 0)),
        ],
        out_specs=pl.BlockSpec((None, out_ch, HW), lambda b: (b, 0, 0)),
        compiler_params=pltpu.CompilerParams(
            dimension_semantics=("parallel",)),
    )(y_pre, stats)

    return y.reshape(B, out_ch, H, W)


# speedup vs baseline: 4.4503x; 1.0345x over previous
import math
import functools

import jax
import jax.numpy as jnp
from jax import lax
from jax.experimental import pallas as pl
from jax.experimental.pallas import tpu as pltpu


def _attn_kernel(cen_ref, wq_ref, wkv_ref, wo_ref, y_ref, st_ref, *,
                 num_heads, hidden, hid8, H, W, inv_sqrt_area):
    hslice = hid8 // num_heads

    cen = cen_ref[...]
    cen_bf = cen.astype(jnp.bfloat16)

    pix = lax.broadcasted_iota(jnp.int32, cen.shape, 1)
    row = lax.shift_right_logical(pix, 5)
    col = lax.bitwise_and(pix, W - 1)

    q_all = jnp.dot(wq_ref[...], cen_bf, preferred_element_type=jnp.float32)

    def _roll(x, s):
        return jnp.roll(x, s, axis=1)

    kbfs, vbfs, rks = [], [], []
    for h in range(num_heads):
        d = (1, 2)[h]
        if d == 1:
            rneg = lambda x: jnp.where(row == 0, _roll(x, -W), _roll(x, W))
            rpos = lambda x: jnp.where(row == H - 1, _roll(x, W), _roll(x, -W))
            cn = jnp.where(col == 0, _roll(cen, -1), _roll(cen, 1))
            cp = jnp.where(col == W - 1, _roll(cen, 1), _roll(cen, -1))
        else:
            rneg = lambda x: jnp.where(
                row == 0, _roll(x, -2 * W),
                jnp.where(row == 1, x, _roll(x, 2 * W)))
            rpos = lambda x: jnp.where(
                row == H - 2, x,
                jnp.where(row == H - 1, _roll(x, 2 * W), _roll(x, -2 * W)))
            cn = jnp.where(col == 0, _roll(cen, -2),
                           jnp.where(col == 1, cen, _roll(cen, 2)))
            cp = jnp.where(col == W - 2, cen,
                           jnp.where(col == W - 1, _roll(cen, 2),
                                     _roll(cen, -2)))
        imgs = (rneg(cn), rneg(cen), rneg(cp), cn, cp,
                rpos(cn), rpos(cen), rpos(cp))
        sur = jnp.concatenate([im - cen for im in imgs],
                              axis=0).astype(jnp.bfloat16)
        kv = jnp.dot(wkv_ref[h], sur, preferred_element_type=jnp.float32)
        k = kv[:hid8]
        rks.append(lax.rsqrt(jnp.maximum(
            jnp.sum(k * k, axis=-1, keepdims=True), 1e-24)))
        kbfs.append(k.astype(jnp.bfloat16))
        vbfs.append(kv[hid8:].astype(jnp.bfloat16))

    wps, vs = [], []
    for n in range(num_heads):
        lo = n * hslice
        k = jnp.concatenate([kb[lo:lo + hslice] for kb in kbfs], axis=0)
        rk = jnp.concatenate([r[lo:lo + hslice] for r in rks], axis=0)
        q = q_all[n * hidden:(n + 1) * hidden]

        qn = (q * (lax.rsqrt(jnp.maximum(
            jnp.sum(q * q, axis=-1, keepdims=True), 1e-24)) * inv_sqrt_area)
              ).astype(jnp.bfloat16)

        s = lax.dot_general(k, qn, (((1,), (1,)), ((), ())),
                            preferred_element_type=jnp.float32) * rk

        cnt = hidden * hid8
        mu = jnp.sum(jnp.sum(s, axis=0, keepdims=True),
                     axis=1, keepdims=True) / cnt
        c = s - mu
        var = jnp.sum(jnp.sum(c * c, axis=0, keepdims=True),
                      axis=1, keepdims=True) / cnt
        c = c * lax.rsqrt(var + 1e-5)

        c = c - jnp.max(c, axis=0, keepdims=True)
        e = jnp.exp(c)
        p = (e / jnp.sum(e, axis=0, keepdims=True)).astype(jnp.bfloat16)

        wps.append(lax.dot_general(p, wo_ref[n], (((1,), (1,)), ((), ())),
                                   preferred_element_type=jnp.float32
                                   ).astype(jnp.bfloat16))
        vs.append(jnp.concatenate(
            [vb[lo:lo + hslice] for vb in vbfs], axis=0))

    wp_all = jnp.concatenate(wps, axis=0)
    v_all = jnp.concatenate(vs, axis=0)
    y = lax.dot_general(wp_all, v_all, (((0,), (0,)), ((), ())),
                        preferred_element_type=jnp.float32)

    y_ref[...] = y
    st_ref[...] = jnp.concatenate(
        [jnp.sum(y, axis=1, keepdims=True),
         jnp.sum(y * y, axis=1, keepdims=True)], axis=1)


def _bn_relu_kernel(y_ref, st_ref, o_ref, *, count):
    tot = jnp.sum(st_ref[...], axis=0)
    inv = 1.0 / count
    mu = tot[:, 0:1] * inv
    var = tot[:, 1:2] * inv - mu * mu
    scale = lax.rsqrt(var + 1e-5)
    o_ref[...] = jnp.maximum((y_ref[...] - mu[None]) * scale[None], 0.0)


def kernel(wq, wk, wv, wo, cen):
    B, C, H, W = cen.shape
    NH, hidden = wq.shape[0], wq.shape[1]
    hid8 = wk.shape[1]
    tra = NH * hidden
    out_ch = wo.shape[0]
    HW = H * W

    cen_flat = cen.astype(jnp.float32).reshape(B, C, HW)

    wq_perm = wq.transpose(1, 0, 2).reshape(tra, C).astype(jnp.bfloat16)
    wkv = jnp.stack([jnp.concatenate([wk[h], wv[h]], axis=0)
                     for h in range(NH)]).astype(jnp.bfloat16)
    wo_r = wo.reshape(out_ch, NH, hidden).transpose(1, 0, 2).astype(
        jnp.bfloat16)

    attn = functools.partial(
        _attn_kernel, num_heads=NH, hidden=hidden, hid8=hid8, H=H, W=W,
        inv_sqrt_area=1.0 / math.sqrt(HW))

    y_pre, stats = pl.pallas_call(
        attn,
        out_shape=(jax.ShapeDtypeStruct((B, out_ch, HW), jnp.float32),
                   jax.ShapeDtypeStruct((B, out_ch, 2), jnp.float32)),
        grid=(B,),
        in_specs=[
            pl.BlockSpec((None, C, HW), lambda b: (b, 0, 0)),
            pl.BlockSpec((tra, C), lambda b: (0, 0)),
            pl.BlockSpec((NH, 2 * hid8, 8 * C), lambda b: (0, 0, 0)),
            pl.BlockSpec((NH, out_ch, hidden), lambda b: (0, 0, 0)),
        ],
        out_specs=(pl.BlockSpec((None, out_ch, HW), lambda b: (b, 0, 0)),
                   pl.BlockSpec((None, out_ch, 2), lambda b: (b, 0, 0))),
        compiler_params=pltpu.CompilerParams(
            dimension_semantics=("parallel",)),
    )(cen_flat, wq_perm, wkv, wo_r)

    bblk = 8
    bn = functools.partial(_bn_relu_kernel, count=float(B * HW))
    y = pl.pallas_call(
        bn,
        out_shape=jax.ShapeDtypeStruct((B, out_ch, HW), jnp.float32),
        grid=(B // bblk,),
        in_specs=[
            pl.BlockSpec((bblk, out_ch, HW), lambda b: (b, 0, 0)),
            pl.BlockSpec((B, out_ch, 2), lambda b: (0, 0, 0)),
        ],
        out_specs=pl.BlockSpec((bblk, out_ch, HW), lambda b: (b, 0, 0)),
        compiler_params=pltpu.CompilerParams(
            dimension_semantics=("parallel",)),
    )(y_pre, stats)

    return y.reshape(B, out_ch, H, W)


# speedup vs baseline: 4.8624x; 1.0926x over previous
import math
import functools

import jax
import jax.numpy as jnp
from jax import lax
from jax.experimental import pallas as pl
from jax.experimental.pallas import tpu as pltpu


def _one_batch(cen, wq_ref, wkv_ref, wo_ref, *,
               num_heads, hidden, hid8, H, W, inv_sqrt_area):
    hslice = hid8 // num_heads
    cen_bf = cen.astype(jnp.bfloat16)

    pix = lax.broadcasted_iota(jnp.int32, cen.shape, 1)
    row = lax.shift_right_logical(pix, 5)
    col = lax.bitwise_and(pix, W - 1)

    q_all = jnp.dot(wq_ref[...], cen_bf, preferred_element_type=jnp.float32)

    def _roll(x, s):
        return jnp.roll(x, s, axis=1)

    kbfs, vbfs = [], []
    for h in range(num_heads):
        d = (1, 2)[h]
        if d == 1:
            rneg = lambda x: jnp.where(row == 0, _roll(x, -W), _roll(x, W))
            rpos = lambda x: jnp.where(row == H - 1, _roll(x, W), _roll(x, -W))
            cn = jnp.where(col == 0, _roll(cen, -1), _roll(cen, 1))
            cp = jnp.where(col == W - 1, _roll(cen, 1), _roll(cen, -1))
        else:
            rneg = lambda x: jnp.where(
                row == 0, _roll(x, -2 * W),
                jnp.where(row == 1, x, _roll(x, 2 * W)))
            rpos = lambda x: jnp.where(
                row == H - 2, x,
                jnp.where(row == H - 1, _roll(x, 2 * W), _roll(x, -2 * W)))
            cn = jnp.where(col == 0, _roll(cen, -2),
                           jnp.where(col == 1, cen, _roll(cen, 2)))
            cp = jnp.where(col == W - 2, cen,
                           jnp.where(col == W - 1, _roll(cen, 2),
                                     _roll(cen, -2)))
        imgs = (rneg(cn), rneg(cen), rneg(cp), cn, cp,
                rpos(cn), rpos(cen), rpos(cp))
        sur = jnp.concatenate([im - cen for im in imgs],
                              axis=0).astype(jnp.bfloat16)
        kv = jnp.dot(wkv_ref[h], sur, preferred_element_type=jnp.float32)
        k = kv[:hid8]
        kn = k * lax.rsqrt(jnp.maximum(
            jnp.sum(k * k, axis=-1, keepdims=True), 1e-24))
        kbfs.append(kn.astype(jnp.bfloat16))
        vbfs.append(kv[hid8:].astype(jnp.bfloat16))

    outs = []
    for n in range(num_heads):
        lo = n * hslice
        kn = jnp.concatenate([kb[lo:lo + hslice] for kb in kbfs], axis=0)
        v = jnp.concatenate([vb[lo:lo + hslice] for vb in vbfs], axis=0)
        q = q_all[n * hidden:(n + 1) * hidden]

        qn = (q * (lax.rsqrt(jnp.maximum(
            jnp.sum(q * q, axis=-1, keepdims=True), 1e-24)) * inv_sqrt_area)
              ).astype(jnp.bfloat16)

        s = lax.dot_general(qn, kn, (((1,), (1,)), ((), ())),
                            preferred_element_type=jnp.float32)

        cnt = hidden * hid8
        mu = jnp.sum(jnp.sum(s, axis=-1, keepdims=True),
                     axis=0, keepdims=True) / cnt
        c = s - mu
        var = jnp.sum(jnp.sum(c * c, axis=-1, keepdims=True),
                      axis=0, keepdims=True) / cnt
        c = c * lax.rsqrt(var + 1e-5)

        c = c - jnp.max(c, axis=-1, keepdims=True)
        e = jnp.exp(c)
        p = (e / jnp.sum(e, axis=-1, keepdims=True)).astype(jnp.bfloat16)

        outs.append(jnp.dot(p, v, preferred_element_type=jnp.float32))

    o_all = jnp.concatenate(outs, axis=0)
    y = jnp.dot(wo_ref[...], o_all, preferred_element_type=jnp.float32)
    st = jnp.concatenate(
        [jnp.sum(y, axis=1, keepdims=True),
         jnp.sum(y * y, axis=1, keepdims=True)], axis=1)
    return y, st


def _attn_kernel(cen_ref, wq_ref, wkv_ref, wo_ref, y_ref, st_ref, *,
                 gblk, **kw):
    for g in range(gblk):
        y, st = _one_batch(cen_ref[g], wq_ref, wkv_ref, wo_ref, **kw)
        y_ref[g] = y
        st_ref[g] = st


def _bn_relu_kernel(y_ref, st_ref, o_ref, *, count):
    tot = jnp.sum(st_ref[...], axis=0)
    inv = 1.0 / count
    mu = tot[:, 0:1] * inv
    var = tot[:, 1:2] * inv - mu * mu
    scale = lax.rsqrt(var + 1e-5)
    o_ref[...] = jnp.maximum((y_ref[...] - mu) * scale, 0.0)


def kernel(wq, wk, wv, wo, cen):
    B, C, H, W = cen.shape
    NH, hidden = wq.shape[0], wq.shape[1]
    hid8 = wk.shape[1]
    tra = NH * hidden
    out_ch = wo.shape[0]
    HW = H * W

    cen_flat = cen.astype(jnp.float32).reshape(B, C, HW)

    wq_perm = wq.transpose(1, 0, 2).reshape(tra, C).astype(jnp.bfloat16)
    wkv = jnp.stack([jnp.concatenate([wk[h], wv[h]], axis=0)
                     for h in range(NH)]).astype(jnp.bfloat16)

    gblk = 2
    attn = functools.partial(
        _attn_kernel, gblk=gblk, num_heads=NH, hidden=hidden, hid8=hid8,
        H=H, W=W, inv_sqrt_area=1.0 / math.sqrt(HW))

    y_pre, stats = pl.pallas_call(
        attn,
        out_shape=(jax.ShapeDtypeStruct((B, out_ch, HW), jnp.float32),
                   jax.ShapeDtypeStruct((B, out_ch, 2), jnp.float32)),
        grid=(B // gblk,),
        in_specs=[
            pl.BlockSpec((gblk, C, HW), lambda b: (b, 0, 0)),
            pl.BlockSpec((tra, C), lambda b: (0, 0)),
            pl.BlockSpec((NH, 2 * hid8, 8 * C), lambda b: (0, 0, 0)),
            pl.BlockSpec((out_ch, tra), lambda b: (0, 0)),
        ],
        out_specs=(pl.BlockSpec((gblk, out_ch, HW), lambda b: (b, 0, 0)),
                   pl.BlockSpec((gblk, out_ch, 2), lambda b: (b, 0, 0))),
        compiler_params=pltpu.CompilerParams(
            dimension_semantics=("parallel",)),
    )(cen_flat, wq_perm, wkv, wo.astype(jnp.float32))

    bblk = 8
    bn = functools.partial(_bn_relu_kernel, count=float(B * HW))
    y = pl.pallas_call(
        bn,
        out_shape=jax.ShapeDtypeStruct((B, out_ch, HW), jnp.float32),
        grid=(B // bblk,),
        in_specs=[
            pl.BlockSpec((bblk, out_ch, HW), lambda b: (b, 0, 0)),
            pl.BlockSpec((B, out_ch, 2), lambda b: (0, 0, 0)),
        ],
        out_specs=pl.BlockSpec((bblk, out_ch, HW), lambda b: (b, 0, 0)),
        compiler_params=pltpu.CompilerParams(
            dimension_semantics=("parallel",)),
    )(y_pre, stats)

    return y.reshape(B, out_ch, H, W)


# speedup vs baseline: 5.8291x; 1.1988x over previous
import math
import functools

import jax
import jax.numpy as jnp
from jax import lax
from jax.experimental import pallas as pl
from jax.experimental.pallas import tpu as pltpu


def _attn_kernel(cen_ref, wq_ref, wkv_ref, wo_ref, y_ref, st_ref, *,
                 gblk, num_heads, hidden, hid8, H, W, inv_sqrt_area):
    hslice = hid8 // num_heads

    def _roll(x, s):
        return jnp.roll(x, s, axis=1)

    cens = [cen_ref[g] for g in range(gblk)]
    pix = lax.broadcasted_iota(jnp.int32, cens[0].shape, 1)
    row = lax.shift_right_logical(pix, 5)
    col = lax.bitwise_and(pix, W - 1)

    q_alls = [jnp.dot(wq_ref[...], c.astype(jnp.bfloat16),
                      preferred_element_type=jnp.float32) for c in cens]

    kvs = [[None] * num_heads for _ in range(gblk)]
    for h in range(num_heads):
        d = (1, 2)[h]
        for g in range(gblk):
            cen = cens[g]
            if d == 1:
                rneg = lambda x: jnp.where(row == 0, _roll(x, -W),
                                           _roll(x, W))
                rpos = lambda x: jnp.where(row == H - 1, _roll(x, W),
                                           _roll(x, -W))
                cn = jnp.where(col == 0, _roll(cen, -1), _roll(cen, 1))
                cp = jnp.where(col == W - 1, _roll(cen, 1), _roll(cen, -1))
            else:
                rneg = lambda x: jnp.where(
                    row == 0, _roll(x, -2 * W),
                    jnp.where(row == 1, x, _roll(x, 2 * W)))
                rpos = lambda x: jnp.where(
                    row == H - 2, x,
                    jnp.where(row == H - 1, _roll(x, 2 * W),
                              _roll(x, -2 * W)))
                cn = jnp.where(col == 0, _roll(cen, -2),
                               jnp.where(col == 1, cen, _roll(cen, 2)))
                cp = jnp.where(col == W - 2, cen,
                               jnp.where(col == W - 1, _roll(cen, 2),
                                         _roll(cen, -2)))
            imgs = (rneg(cn), rneg(cen), rneg(cp), cn, cp,
                    rpos(cn), rpos(cen), rpos(cp))
            sur = jnp.concatenate([im - cen for im in imgs],
                                  axis=0).astype(jnp.bfloat16)
            kvs[g][h] = jnp.dot(wkv_ref[h], sur,
                                preferred_element_type=jnp.float32)

    kbfs, vbfs = [], []
    for g in range(gblk):
        kb, vb = [], []
        for h in range(num_heads):
            k = kvs[g][h][:hid8]
            kn = k * lax.rsqrt(jnp.maximum(
                jnp.sum(k * k, axis=-1, keepdims=True), 1e-24))
            kb.append(kn.astype(jnp.bfloat16))
            vb.append(kvs[g][h][hid8:].astype(jnp.bfloat16))
        kbfs.append(kb)
        vbfs.append(vb)

    cnt = hidden * hid8
    ss, vsel_ = [], []
    for g in range(gblk):
        for n in range(num_heads):
            lo = n * hslice
            kn = jnp.concatenate([kb[lo:lo + hslice] for kb in kbfs[g]],
                                 axis=0)
            v = jnp.concatenate([vb[lo:lo + hslice] for vb in vbfs[g]],
                                axis=0)
            q = q_alls[g][n * hidden:(n + 1) * hidden]
            qn = (q * (lax.rsqrt(jnp.maximum(
                jnp.sum(q * q, axis=-1, keepdims=True), 1e-24))
                * inv_sqrt_area)).astype(jnp.bfloat16)
            ss.append(lax.dot_general(qn, kn, (((1,), (1,)), ((), ())),
                                      preferred_element_type=jnp.float32))
            vsel_.append(v)

    ps = []
    for s in ss:
        tot = jnp.sum(jnp.sum(s, axis=-1, keepdims=True),
                      axis=0, keepdims=True)
        tot2 = jnp.sum(jnp.sum(s * s, axis=-1, keepdims=True),
                       axis=0, keepdims=True)
        mu = tot / cnt
        var = tot2 / cnt - mu * mu
        e = jnp.exp((s - mu) * lax.rsqrt(var + 1e-5))
        ps.append((e / jnp.sum(e, axis=-1, keepdims=True)
                   ).astype(jnp.bfloat16))

    for g in range(gblk):
        outs = [jnp.dot(ps[g * num_heads + n], vsel_[g * num_heads + n],
                        preferred_element_type=jnp.float32)
                for n in range(num_heads)]
        o_all = jnp.concatenate(outs, axis=0)
        y = jnp.dot(wo_ref[...], o_all, preferred_element_type=jnp.float32)
        y_ref[g] = y
        st_ref[g] = jnp.concatenate(
            [jnp.sum(y, axis=1, keepdims=True),
             jnp.sum(y * y, axis=1, keepdims=True)], axis=1)


def _bn_relu_kernel(y_ref, st_ref, o_ref, *, count):
    tot = jnp.sum(st_ref[...], axis=0)
    inv = 1.0 / count
    mu = tot[:, 0:1] * inv
    var = tot[:, 1:2] * inv - mu * mu
    scale = lax.rsqrt(var + 1e-5)
    o_ref[...] = jnp.maximum((y_ref[...] - mu) * scale, 0.0)


def kernel(wq, wk, wv, wo, cen):
    B, C, H, W = cen.shape
    NH, hidden = wq.shape[0], wq.shape[1]
    hid8 = wk.shape[1]
    tra = NH * hidden
    out_ch = wo.shape[0]
    HW = H * W

    cen_flat = cen.astype(jnp.float32).reshape(B, C, HW)

    wq_perm = wq.transpose(1, 0, 2).reshape(tra, C).astype(jnp.bfloat16)
    wkv = jnp.stack([jnp.concatenate([wk[h], wv[h]], axis=0)
                     for h in range(NH)]).astype(jnp.bfloat16)

    gblk = 2
    attn = functools.partial(
        _attn_kernel, gblk=gblk, num_heads=NH, hidden=hidden, hid8=hid8,
        H=H, W=W, inv_sqrt_area=1.0 / math.sqrt(HW))

    y_pre, stats = pl.pallas_call(
        attn,
        out_shape=(jax.ShapeDtypeStruct((B, out_ch, HW), jnp.float32),
                   jax.ShapeDtypeStruct((B, out_ch, 2), jnp.float32)),
        grid=(B // gblk,),
        in_specs=[
            pl.BlockSpec((gblk, C, HW), lambda b: (b, 0, 0)),
            pl.BlockSpec((tra, C), lambda b: (0, 0)),
            pl.BlockSpec((NH, 2 * hid8, 8 * C), lambda b: (0, 0, 0)),
            pl.BlockSpec((out_ch, tra), lambda b: (0, 0)),
        ],
        out_specs=(pl.BlockSpec((gblk, out_ch, HW), lambda b: (b, 0, 0)),
                   pl.BlockSpec((gblk, out_ch, 2), lambda b: (b, 0, 0))),
        compiler_params=pltpu.CompilerParams(
            dimension_semantics=("parallel",)),
    )(cen_flat, wq_perm, wkv, wo.astype(jnp.float32))

    bblk = 8
    bn = functools.partial(_bn_relu_kernel, count=float(B * HW))
    y = pl.pallas_call(
        bn,
        out_shape=jax.ShapeDtypeStruct((B, out_ch, HW), jnp.float32),
        grid=(B // bblk,),
        in_specs=[
            pl.BlockSpec((bblk, out_ch, HW), lambda b: (b, 0, 0)),
            pl.BlockSpec((B, out_ch, 2), lambda b: (0, 0, 0)),
        ],
        out_specs=pl.BlockSpec((bblk, out_ch, HW), lambda b: (b, 0, 0)),
        compiler_params=pltpu.CompilerParams(
            dimension_semantics=("parallel",)),
    )(y_pre, stats)

    return y.reshape(B, out_ch, H, W)


# speedup vs baseline: 6.0767x; 1.0425x over previous
import math
import functools

import jax
import jax.numpy as jnp
from jax import lax
from jax.experimental import pallas as pl
from jax.experimental.pallas import tpu as pltpu


def _attn_kernel(cen_ref, wq_ref, wkv_ref, wo_ref, y_ref, st_ref, *,
                 gblk, num_heads, hidden, hid8, H, W, inv_sqrt_area):
    hslice = hid8 // num_heads

    def _roll(x, s):
        return jnp.roll(x, s, axis=1)

    cens = [cen_ref[g] for g in range(gblk)]
    pix = lax.broadcasted_iota(jnp.int32, cens[0].shape, 1)
    row = lax.shift_right_logical(pix, 5)
    col = lax.bitwise_and(pix, W - 1)

    q_alls = [jnp.dot(wq_ref[...], c.astype(jnp.bfloat16),
                      preferred_element_type=jnp.float32) for c in cens]

    kbfs = [[None] * num_heads for _ in range(gblk)]
    vbfs = [[None] * num_heads for _ in range(gblk)]
    for h in range(num_heads):
        d = (1, 2)[h]
        for g in range(gblk):
            cen = cens[g]
            if d == 1:
                rneg = lambda x: jnp.where(row == 0, _roll(x, -W),
                                           _roll(x, W))
                rpos = lambda x: jnp.where(row == H - 1, _roll(x, W),
                                           _roll(x, -W))
                cn = jnp.where(col == 0, _roll(cen, -1), _roll(cen, 1))
                cp = jnp.where(col == W - 1, _roll(cen, 1), _roll(cen, -1))
            else:
                rneg = lambda x: jnp.where(
                    row == 0, _roll(x, -2 * W),
                    jnp.where(row == 1, x, _roll(x, 2 * W)))
                rpos = lambda x: jnp.where(
                    row == H - 2, x,
                    jnp.where(row == H - 1, _roll(x, 2 * W),
                              _roll(x, -2 * W)))
                cn = jnp.where(col == 0, _roll(cen, -2),
                               jnp.where(col == 1, cen, _roll(cen, 2)))
                cp = jnp.where(col == W - 2, cen,
                               jnp.where(col == W - 1, _roll(cen, 2),
                                         _roll(cen, -2)))
            imgs = (rneg(cn), rneg(cen), rneg(cp), cn, cp,
                    rpos(cn), rpos(cen), rpos(cp))
            sur = jnp.concatenate([im - cen for im in imgs],
                                  axis=0).astype(jnp.bfloat16)
            kv = jnp.dot(wkv_ref[h], sur,
                         preferred_element_type=jnp.float32)
            k = kv[:hid8]
            kn = k * lax.rsqrt(jnp.maximum(
                jnp.sum(k * k, axis=-1, keepdims=True), 1e-24))
            kbfs[g][h] = kn.astype(jnp.bfloat16)
            vbfs[g][h] = kv[hid8:].astype(jnp.bfloat16)

    cnt = hidden * hid8
    ss, vsel_ = [], []
    for g in range(gblk):
        for n in range(num_heads):
            lo = n * hslice
            kn = jnp.concatenate([kb[lo:lo + hslice] for kb in kbfs[g]],
                                 axis=0)
            v = jnp.concatenate([vb[lo:lo + hslice] for vb in vbfs[g]],
                                axis=0)
            q = q_alls[g][n * hidden:(n + 1) * hidden]
            qn = (q * (lax.rsqrt(jnp.maximum(
                jnp.sum(q * q, axis=-1, keepdims=True), 1e-24))
                * inv_sqrt_area)).astype(jnp.bfloat16)
            ss.append(lax.dot_general(qn, kn, (((1,), (1,)), ((), ())),
                                      preferred_element_type=jnp.float32))
            vsel_.append(v)

    ps = []
    for s in ss:
        tot = jnp.sum(jnp.sum(s, axis=-1, keepdims=True),
                      axis=0, keepdims=True)
        tot2 = jnp.sum(jnp.sum(s * s, axis=-1, keepdims=True),
                       axis=0, keepdims=True)
        mu = tot / cnt
        var = tot2 / cnt - mu * mu
        e = jnp.exp((s - mu) * lax.rsqrt(var + 1e-5))
        ps.append((e / jnp.sum(e, axis=-1, keepdims=True)
                   ).astype(jnp.bfloat16))

    for g in range(gblk):
        outs = [jnp.dot(ps[g * num_heads + n], vsel_[g * num_heads + n],
                        preferred_element_type=jnp.float32)
                for n in range(num_heads)]
        o_all = jnp.concatenate(outs, axis=0)
        y = jnp.dot(wo_ref[...], o_all, preferred_element_type=jnp.float32)
        y_ref[g] = y
        st_ref[g] = jnp.concatenate(
            [jnp.sum(y, axis=1, keepdims=True),
             jnp.sum(y * y, axis=1, keepdims=True)], axis=1)


def _bn_relu_kernel(y_ref, st_ref, o_ref, *, count):
    tot = jnp.sum(st_ref[...], axis=0)
    inv = 1.0 / count
    mu = tot[:, 0:1] * inv
    var = tot[:, 1:2] * inv - mu * mu
    scale = lax.rsqrt(var + 1e-5)
    o_ref[...] = jnp.maximum((y_ref[...] - mu) * scale, 0.0)


def kernel(wq, wk, wv, wo, cen):
    B, C, H, W = cen.shape
    NH, hidden = wq.shape[0], wq.shape[1]
    hid8 = wk.shape[1]
    tra = NH * hidden
    out_ch = wo.shape[0]
    HW = H * W

    cen_flat = cen.astype(jnp.float32).reshape(B, C, HW)

    wq_perm = wq.transpose(1, 0, 2).reshape(tra, C).astype(jnp.bfloat16)
    wkv = jnp.stack([jnp.concatenate([wk[h], wv[h]], axis=0)
                     for h in range(NH)]).astype(jnp.bfloat16)

    gblk = min(4, B)
    attn = functools.partial(
        _attn_kernel, gblk=gblk, num_heads=NH, hidden=hidden, hid8=hid8,
        H=H, W=W, inv_sqrt_area=1.0 / math.sqrt(HW))

    y_pre, stats = pl.pallas_call(
        attn,
        out_shape=(jax.ShapeDtypeStruct((B, out_ch, HW), jnp.float32),
                   jax.ShapeDtypeStruct((B, out_ch, 2), jnp.float32)),
        grid=(B // gblk,),
        in_specs=[
            pl.BlockSpec((gblk, C, HW), lambda b: (b, 0, 0)),
            pl.BlockSpec((tra, C), lambda b: (0, 0)),
            pl.BlockSpec((NH, 2 * hid8, 8 * C), lambda b: (0, 0, 0)),
            pl.BlockSpec((out_ch, tra), lambda b: (0, 0)),
        ],
        out_specs=(pl.BlockSpec((gblk, out_ch, HW), lambda b: (b, 0, 0)),
                   pl.BlockSpec((gblk, out_ch, 2), lambda b: (b, 0, 0))),
        compiler_params=pltpu.CompilerParams(
            dimension_semantics=("parallel",)),
    )(cen_flat, wq_perm, wkv, wo.astype(jnp.float32))

    bblk = min(16, B)
    bn = functools.partial(_bn_relu_kernel, count=float(B * HW))
    y = pl.pallas_call(
        bn,
        out_shape=jax.ShapeDtypeStruct((B, out_ch, HW), jnp.float32),
        grid=(B // bblk,),
        in_specs=[
            pl.BlockSpec((bblk, out_ch, HW), lambda b: (b, 0, 0)),
            pl.BlockSpec((B, out_ch, 2), lambda b: (0, 0, 0)),
        ],
        out_specs=pl.BlockSpec((bblk, out_ch, HW), lambda b: (b, 0, 0)),
        compiler_params=pltpu.CompilerParams(
            dimension_semantics=("parallel",)),
    )(y_pre, stats)

    return y.reshape(B, out_ch, H, W)
